# baseline (speedup 1.0000x reference)
import jax
import jax.numpy as jnp
from jax.experimental import pallas as pl
from jax.experimental.pallas import tpu as pltpu
from jax.experimental.pallas import tpu_sc as plsc

_LANES = 16
_ROWS_PER_BLOCK = 128


def _block_body(a_vmem, b_vmem, o_vmem):
    @pl.loop(0, _ROWS_PER_BLOCK)
    def _(r):
        row = pl.ds(r, 1)
        for k in range(8):
            dst = pl.ds(k * _LANES, _LANES)
            if k < 6:
                src_a = a_vmem.at[row, pl.ds(k * _LANES, _LANES)][...]
            if k >= 2:
                src_b = b_vmem.at[row, pl.ds((k - 2) * _LANES, _LANES)][...]
            if k < 2:
                o_vmem.at[row, dst][...] = src_a
            elif k < 6:
                o_vmem.at[row, dst][...] = src_a + src_b
            else:
                o_vmem.at[row, dst][...] = src_b


def kernel(a, b):
    n = a.shape[0]
    mesh = plsc.VectorSubcoreMesh(core_axis_name="core",
                                  subcore_axis_name="subcore")

    @pl.kernel(out_type=jax.ShapeDtypeStruct((n, 128), a.dtype), mesh=mesh)
    def run(a_hbm, b_hbm, o_hbm):
        pltpu.emit_pipeline(
            _block_body,
            grid=(n // _ROWS_PER_BLOCK,),
            in_specs=[
                pl.BlockSpec((_ROWS_PER_BLOCK, 96), lambda i: (i, 0)),
                pl.BlockSpec((_ROWS_PER_BLOCK, 96), lambda i: (i, 0)),
            ],
            out_specs=[pl.BlockSpec((_ROWS_PER_BLOCK, 128), lambda i: (i, 0))],
            core_axis_name=("core", "subcore"),
            dimension_semantics=(pltpu.PARALLEL,),
        )(a_hbm, b_hbm, o_hbm)

    return run(a, b)


# speedup vs baseline: 1.5111x; 1.5111x over previous
import jax
import jax.numpy as jnp
from jax.experimental import pallas as pl
from jax.experimental.pallas import tpu as pltpu
from jax.experimental.pallas import tpu_sc as plsc

_LANES = 16
_ROWS_PER_BLOCK = 128


def _block_body(a_vmem, b_vmem, o_vmem):
    @plsc.parallel_loop(0, _ROWS_PER_BLOCK, unroll=4)
    def _(r):
        row = pl.ds(r, 1)
        for k in range(8):
            dst = pl.ds(k * _LANES, _LANES)
            if k < 6:
                src_a = a_vmem.at[row, pl.ds(k * _LANES, _LANES)][...]
            if k >= 2:
                src_b = b_vmem.at[row, pl.ds((k - 2) * _LANES, _LANES)][...]
            if k < 2:
                o_vmem.at[row, dst][...] = src_a
            elif k < 6:
                o_vmem.at[row, dst][...] = src_a + src_b
            else:
                o_vmem.at[row, dst][...] = src_b


def kernel(a, b):
    n = a.shape[0]
    mesh = plsc.VectorSubcoreMesh(core_axis_name="core",
                                  subcore_axis_name="subcore")

    @pl.kernel(out_type=jax.ShapeDtypeStruct((n, 128), a.dtype), mesh=mesh)
    def run(a_hbm, b_hbm, o_hbm):
        pltpu.emit_pipeline(
            _block_body,
            grid=(n // _ROWS_PER_BLOCK,),
            in_specs=[
                pl.BlockSpec((_ROWS_PER_BLOCK, 96), lambda i: (i, 0)),
                pl.BlockSpec((_ROWS_PER_BLOCK, 96), lambda i: (i, 0)),
            ],
            out_specs=[pl.BlockSpec((_ROWS_PER_BLOCK, 128), lambda i: (i, 0))],
            core_axis_name=("core", "subcore"),
            dimension_semantics=(pltpu.PARALLEL,),
        )(a_hbm, b_hbm, o_hbm)

    return run(a, b)


# speedup vs baseline: 1.5137x; 1.0017x over previous
import jax
import jax.numpy as jnp
from jax.experimental import pallas as pl
from jax.experimental.pallas import tpu as pltpu
from jax.experimental.pallas import tpu_sc as plsc

_LANES = 16
_ROWS_PER_BLOCK = 128


def _block_body(a_vmem, b_vmem, o_vmem):
    @plsc.parallel_loop(0, _ROWS_PER_BLOCK, unroll=8)
    def _(r):
        row = pl.ds(r, 1)
        for k in range(8):
            dst = pl.ds(k * _LANES, _LANES)
            if k < 6:
                src_a = a_vmem.at[row, pl.ds(k * _LANES, _LANES)][...]
            if k >= 2:
                src_b = b_vmem.at[row, pl.ds((k - 2) * _LANES, _LANES)][...]
            if k < 2:
                o_vmem.at[row, dst][...] = src_a
            elif k < 6:
                o_vmem.at[row, dst][...] = src_a + src_b
            else:
                o_vmem.at[row, dst][...] = src_b


def kernel(a, b):
    n = a.shape[0]
    mesh = plsc.VectorSubcoreMesh(core_axis_name="core",
                                  subcore_axis_name="subcore")

    @pl.kernel(out_type=jax.ShapeDtypeStruct((n, 128), a.dtype), mesh=mesh)
    def run(a_hbm, b_hbm, o_hbm):
        pltpu.emit_pipeline(
            _block_body,
            grid=(n // _ROWS_PER_BLOCK,),
            in_specs=[
                pl.BlockSpec((_ROWS_PER_BLOCK, 96), lambda i: (i, 0)),
                pl.BlockSpec((_ROWS_PER_BLOCK, 96), lambda i: (i, 0)),
            ],
            out_specs=[pl.BlockSpec((_ROWS_PER_BLOCK, 128), lambda i: (i, 0))],
            core_axis_name=("core", "subcore"),
            dimension_semantics=(pltpu.PARALLEL,),
        )(a_hbm, b_hbm, o_hbm)

    return run(a, b)
